# Initial kernel scaffold; baseline (speedup 1.0000x reference)
import jax
import jax.numpy as jnp
from jax.experimental import pallas as pl


def kernel(scores, coords, batch_ids):
    raise NotImplementedError("write your pallas kernel here")


# speedup vs baseline: 2.5827x; 2.5827x over previous
import jax
import jax.numpy as jnp
from jax.experimental import pallas as pl

_QB = 32
_K = 16
_KN = 8
_BIG = 3.0e38
_IIBIG = 2**30


def _knn_loss_kernel(q_ref, qs_ref, c_ref, s_ref, pos_ref, neg_ref, sm_ref):
    i = pl.program_id(0)

    n = c_ref.shape[1]
    qx = q_ref[:, 0:1]
    qy = q_ref[:, 1:2]
    qz = q_ref[:, 2:3]
    dx = qx - c_ref[0:1, :]
    dy = qy - c_ref[1:2, :]
    dz = qz - c_ref[2:3, :]
    d2 = dx * dx + dy * dy + dz * dz

    iota = jax.lax.broadcasted_iota(jnp.int32, (q_ref.shape[0], n), 1)
    scores_b = s_ref[0:1, :]
    qs = qs_ref[:, 0:1]

    sum_pos = jnp.zeros_like(qs)
    sum_neg = jnp.zeros_like(qs)
    sum8 = jnp.zeros_like(qs)

    for t in range(_K):
        m = jnp.min(d2, axis=1, keepdims=True)
        eqm = d2 == m
        sel_i = jnp.min(jnp.where(eqm, iota, _IIBIG), axis=1, keepdims=True)
        msel = eqm & (iota == sel_i)
        s = jnp.sum(jnp.where(msel, scores_b, 0.0), axis=1, keepdims=True)
        d2 = jnp.where(msel, _BIG, d2)
        logit = 2.0 * (1.0 - jnp.abs(qs - s))
        if t < _KN:
            sum_pos += -jnp.log(jax.nn.sigmoid(logit) + 1e-8)
            sum8 += s
        else:
            sum_neg += -jnp.log(1.0 - jax.nn.sigmoid(logit) + 1e-8)

    sm = (qs - sum8 * (1.0 / _KN)) ** 2

    @pl.when(i == 0)
    def _init():
        pos_ref[...] = jnp.zeros_like(pos_ref)
        neg_ref[...] = jnp.zeros_like(neg_ref)
        sm_ref[...] = jnp.zeros_like(sm_ref)

    pos_ref[...] += jnp.sum(sum_pos)
    neg_ref[...] += jnp.sum(sum_neg)
    sm_ref[...] += jnp.sum(sm)


def kernel(scores, coords, batch_ids):
    n = scores.shape[0]
    grid = (n // _QB,)
    acc = jax.ShapeDtypeStruct((1, 128), jnp.float32)
    pos, neg, sm = pl.pallas_call(
        _knn_loss_kernel,
        grid=grid,
        in_specs=[
            pl.BlockSpec((_QB, 3), lambda i: (i, 0)),
            pl.BlockSpec((_QB, 1), lambda i: (i, 0)),
            pl.BlockSpec((3, n), lambda i: (0, 0)),
            pl.BlockSpec((1, n), lambda i: (0, 0)),
        ],
        out_specs=[pl.BlockSpec((1, 128), lambda i: (0, 0))] * 3,
        out_shape=[acc, acc, acc],
    )(coords, scores.reshape(n, 1), coords.T, scores.reshape(1, n))

    denom = jnp.float32(1.0 / (n * _KN))
    loss_pos = pos[0, 0] * denom
    loss_neg = neg[0, 0] * denom
    loss_contrastive = loss_pos + loss_neg
    loss_smoothness = sm[0, 0] * jnp.float32(1.0 / n)
    loss_locality = jnp.asarray(0.0, dtype=jnp.float32)
    total = (1.0 * loss_locality + 0.5 * loss_contrastive
             + 0.2 * loss_smoothness)
    return (total, loss_locality, loss_contrastive, loss_smoothness)


# speedup vs baseline: 3.3935x; 1.3139x over previous
import jax
import jax.numpy as jnp
from jax.experimental import pallas as pl

_QB = 128
_L = 128
_K = 16
_KN = 8
_BIG = 3.0e38
_IIBIG = 2**30


def _knn_loss_kernel(q_ref, qs_ref, c_ref, s_ref, pos_ref, neg_ref, sm_ref):
    i = pl.program_id(0)
    qb = q_ref.shape[0]

    qx = q_ref[:, 0:1][:, :, None]
    qy = q_ref[:, 1:2][:, :, None]
    qz = q_ref[:, 2:3][:, :, None]
    cx = c_ref[0][None]
    cy = c_ref[1][None]
    cz = c_ref[2][None]
    dx = qx - cx
    dy = qy - cy
    dz = qz - cz
    d2 = dx * dx + dy * dy + dz * dz

    gmin0 = jnp.min(d2, axis=2)

    g = c_ref.shape[1]
    iota_g = jax.lax.broadcasted_iota(jnp.int32, (qb, g), 1)
    iota_l = jax.lax.broadcasted_iota(jnp.int32, (qb, _L), 1)
    scores_mat = s_ref[...]
    qs = qs_ref[:, 0:1]

    def body(t, carry):
        gmin, kd, ki, sum_pos, sum_neg, sum8 = carry
        m = jnp.min(gmin, axis=1, keepdims=True)
        gsel_i = jnp.min(jnp.where(gmin == m, iota_g, _IIBIG),
                         axis=1, keepdims=True)
        selg = iota_g == gsel_i
        penal = (1.0 - selg.astype(jnp.float32)) * _BIG
        rowd = jnp.min(d2 + penal[:, :, None], axis=1)
        gidx = gsel_i * _L + iota_l
        valid = (rowd > kd) | ((rowd == kd) & (gidx > ki))
        l_i = jnp.min(jnp.where(valid & (rowd == m), gidx, _IIBIG),
                      axis=1, keepdims=True)
        srow = jnp.dot(selg.astype(jnp.float32), scores_mat,
                       preferred_element_type=jnp.float32)
        s = jnp.sum(jnp.where(gidx == l_i, srow, 0.0),
                    axis=1, keepdims=True)
        valid_new = (rowd > m) | ((rowd == m) & (gidx > l_i))
        newmin = jnp.min(jnp.where(valid_new, rowd, _BIG),
                         axis=1, keepdims=True)
        gmin = jnp.where(selg, newmin, gmin)

        logit = 2.0 * (1.0 - jnp.abs(qs - s))
        sig = jax.nn.sigmoid(logit)
        gpos = -jnp.log(sig + 1e-8)
        gneg = -jnp.log(1.0 - sig + 1e-8)
        wpos = jnp.where(t < _KN, 1.0, 0.0)
        sum_pos = sum_pos + wpos * gpos
        sum_neg = sum_neg + (1.0 - wpos) * gneg
        sum8 = sum8 + wpos * s
        return (gmin, m, l_i, sum_pos, sum_neg, sum8)

    z = jnp.zeros_like(qs)
    init = (gmin0, jnp.full_like(qs, -1.0),
            jnp.full_like(qs, -1, dtype=jnp.int32), z, z, z)
    _, _, _, sum_pos, sum_neg, sum8 = jax.lax.fori_loop(0, _K, body, init)

    sm = (qs - sum8 * (1.0 / _KN)) ** 2

    @pl.when(i == 0)
    def _init():
        pos_ref[...] = jnp.zeros_like(pos_ref)
        neg_ref[...] = jnp.zeros_like(neg_ref)
        sm_ref[...] = jnp.zeros_like(sm_ref)

    pos_ref[...] += jnp.sum(sum_pos)
    neg_ref[...] += jnp.sum(sum_neg)
    sm_ref[...] += jnp.sum(sm)


def kernel(scores, coords, batch_ids):
    n = scores.shape[0]
    g = n // _L
    grid = (n // _QB,)
    acc = jax.ShapeDtypeStruct((1, 128), jnp.float32)
    pos, neg, sm = pl.pallas_call(
        _knn_loss_kernel,
        grid=grid,
        in_specs=[
            pl.BlockSpec((_QB, 3), lambda i: (i, 0)),
            pl.BlockSpec((_QB, 1), lambda i: (i, 0)),
            pl.BlockSpec((3, g, _L), lambda i: (0, 0, 0)),
            pl.BlockSpec((g, _L), lambda i: (0, 0)),
        ],
        out_specs=[pl.BlockSpec((1, 128), lambda i: (0, 0))] * 3,
        out_shape=[acc, acc, acc],
    )(coords, scores.reshape(n, 1), coords.T.reshape(3, g, _L),
      scores.reshape(g, _L))

    denom = jnp.float32(1.0 / (n * _KN))
    loss_pos = pos[0, 0] * denom
    loss_neg = neg[0, 0] * denom
    loss_contrastive = loss_pos + loss_neg
    loss_smoothness = sm[0, 0] * jnp.float32(1.0 / n)
    loss_locality = jnp.asarray(0.0, dtype=jnp.float32)
    total = (1.0 * loss_locality + 0.5 * loss_contrastive
             + 0.2 * loss_smoothness)
    return (total, loss_locality, loss_contrastive, loss_smoothness)
